# Initial kernel scaffold; baseline (speedup 1.0000x reference)
import jax
import jax.numpy as jnp
from jax.experimental import pallas as pl


def kernel(x, projection_mat, emb_weight):
    raise NotImplementedError("write your pallas kernel here")


# speedup vs baseline: 123.4639x; 123.4639x over previous
import functools

import jax
import jax.numpy as jnp
from jax.experimental import pallas as pl
from jax.experimental.pallas import tpu as pltpu

INP_DIM = 1024
EMB_DIM = 64
N_PROJ = 16
NUM_BINS = 20
NCOLS = (NUM_BINS + 1) * N_PROJ

TOK_BLOCK = 512


def _consts():
    resolution = 2.0 / NUM_BINS
    grid = jnp.linspace(-1.0, 1.0, NUM_BINS + 1)[:-1] + 0.5 * resolution
    lower = jnp.concatenate([jnp.full((1,), -3.0, jnp.float32), grid])
    upper = jnp.concatenate([grid, jnp.full((1,), 3.0, jnp.float32)])
    lower = jnp.tile(lower, (N_PROJ,)).reshape(1, NCOLS)
    upper = jnp.tile(upper, (N_PROJ,)).reshape(1, NCOLS)
    p_of_col = jnp.arange(NCOLS, dtype=jnp.int32) // (NUM_BINS + 1)
    expand = (p_of_col[None, :] == jnp.arange(N_PROJ, dtype=jnp.int32)[:, None])
    expand = expand.astype(jnp.float32)
    return lower, upper, expand


def _body(x_ref, pm_ref, low_ref, up_ref, ex_ref, w_ref, out_ref):
    xb = x_ref[...]
    ssq = jnp.sum(xb * xb, axis=1, keepdims=True)
    norm = jnp.maximum(jnp.sqrt(ssq), 1e-12)
    xn = xb / norm
    z = jnp.dot(xn.astype(jnp.bfloat16), pm_ref[...].astype(jnp.bfloat16),
                preferred_element_type=jnp.float32)
    zc = jnp.dot(z, ex_ref[...], preferred_element_type=jnp.float32,
                 precision=jax.lax.Precision.HIGHEST)
    onehot = ((zc > low_ref[...]).astype(jnp.float32)
              - (zc > up_ref[...]).astype(jnp.float32))
    acc = jnp.dot(onehot, w_ref[...], preferred_element_type=jnp.float32,
                  precision=jax.lax.Precision.HIGHEST)
    out_ref[...] = acc * (1.0 / N_PROJ)


@functools.partial(jax.jit, static_argnames=("interpret",))
def kernel(x, projection_mat, emb_weight, interpret=False):
    bs, seq_len, _ = x.shape
    ntok = bs * seq_len
    xf = x.reshape(ntok, INP_DIM)
    lower, upper, expand = _consts()
    grid_n = ntok // TOK_BLOCK
    out = pl.pallas_call(
        _body,
        grid=(grid_n,),
        in_specs=[
            pl.BlockSpec((TOK_BLOCK, INP_DIM), lambda i: (i, 0)),
            pl.BlockSpec((INP_DIM, N_PROJ), lambda i: (0, 0)),
            pl.BlockSpec((1, NCOLS), lambda i: (0, 0)),
            pl.BlockSpec((1, NCOLS), lambda i: (0, 0)),
            pl.BlockSpec((N_PROJ, NCOLS), lambda i: (0, 0)),
            pl.BlockSpec((NCOLS, EMB_DIM), lambda i: (0, 0)),
        ],
        out_specs=pl.BlockSpec((TOK_BLOCK, EMB_DIM), lambda i: (i, 0)),
        out_shape=jax.ShapeDtypeStruct((ntok, EMB_DIM), jnp.float32),
        compiler_params=pltpu.CompilerParams(
            dimension_semantics=("arbitrary",),
        ),
        interpret=interpret,
    )(xf, projection_mat, lower, upper, expand, emb_weight)
    return out.reshape(bs, seq_len, EMB_DIM)


# speedup vs baseline: 177.5662x; 1.4382x over previous
import functools

import jax
import jax.numpy as jnp
from jax.experimental import pallas as pl
from jax.experimental.pallas import tpu as pltpu

INP_DIM = 1024
EMB_DIM = 64
N_PROJ = 16
NUM_BINS = 20
NCOLS = (NUM_BINS + 1) * N_PROJ

TOK_BLOCK = 512


def _consts():
    resolution = 2.0 / NUM_BINS
    grid = jnp.linspace(-1.0, 1.0, NUM_BINS + 1)[:-1] + 0.5 * resolution
    lower = jnp.concatenate([jnp.full((1,), -3.0, jnp.float32), grid])
    upper = jnp.concatenate([grid, jnp.full((1,), 3.0, jnp.float32)])
    lower = jnp.tile(lower, (N_PROJ,)).reshape(1, NCOLS)
    upper = jnp.tile(upper, (N_PROJ,)).reshape(1, NCOLS)
    p_of_col = jnp.arange(NCOLS, dtype=jnp.int32) // (NUM_BINS + 1)
    expand = (p_of_col[None, :] == jnp.arange(N_PROJ, dtype=jnp.int32)[:, None])
    expand = expand.astype(jnp.bfloat16)
    return lower, upper, expand


def _body(x_ref, pm_ref, low_ref, up_ref, ex_ref, w_ref, out_ref):
    xb = x_ref[...]
    ssq = jnp.sum(xb * xb, axis=1, keepdims=True)
    norm = jnp.maximum(jnp.sqrt(ssq), 1e-12)
    xn = xb / norm
    z = jnp.dot(xn.astype(jnp.bfloat16), pm_ref[...].astype(jnp.bfloat16),
                preferred_element_type=jnp.float32)
    ex = ex_ref[...]
    z_hi = z.astype(jnp.bfloat16)
    z_lo = (z - z_hi.astype(jnp.float32)).astype(jnp.bfloat16)
    zc = (jnp.dot(z_hi, ex, preferred_element_type=jnp.float32)
          + jnp.dot(z_lo, ex, preferred_element_type=jnp.float32))
    onehot = ((zc > low_ref[...]).astype(jnp.bfloat16)
              - (zc > up_ref[...]).astype(jnp.bfloat16))
    acc = jnp.dot(onehot, w_ref[...].astype(jnp.bfloat16),
                  preferred_element_type=jnp.float32)
    out_ref[...] = acc * (1.0 / N_PROJ)


@functools.partial(jax.jit, static_argnames=("interpret",))
def kernel(x, projection_mat, emb_weight, interpret=False):
    bs, seq_len, _ = x.shape
    ntok = bs * seq_len
    xf = x.reshape(ntok, INP_DIM)
    lower, upper, expand = _consts()
    grid_n = ntok // TOK_BLOCK
    out = pl.pallas_call(
        _body,
        grid=(grid_n,),
        in_specs=[
            pl.BlockSpec((TOK_BLOCK, INP_DIM), lambda i: (i, 0)),
            pl.BlockSpec((INP_DIM, N_PROJ), lambda i: (0, 0)),
            pl.BlockSpec((1, NCOLS), lambda i: (0, 0)),
            pl.BlockSpec((1, NCOLS), lambda i: (0, 0)),
            pl.BlockSpec((N_PROJ, NCOLS), lambda i: (0, 0)),
            pl.BlockSpec((NCOLS, EMB_DIM), lambda i: (0, 0)),
        ],
        out_specs=pl.BlockSpec((TOK_BLOCK, EMB_DIM), lambda i: (i, 0)),
        out_shape=jax.ShapeDtypeStruct((ntok, EMB_DIM), jnp.float32),
        compiler_params=pltpu.CompilerParams(
            dimension_semantics=("arbitrary",),
        ),
        interpret=interpret,
    )(xf, projection_mat, lower, upper, expand, emb_weight)
    return out.reshape(bs, seq_len, EMB_DIM)
